# baseline (speedup 1.0000x reference)
import functools

import jax
import jax.numpy as jnp
from jax import lax
from jax.experimental import pallas as pl
from jax.experimental.pallas import tpu as pltpu
from jax.experimental.pallas import tpu_sc as plsc

N_REAL = 10000
N_CANDS = 100
M = N_REAL + N_CANDS
MP = 10240
E = 320000
NW = 32
CH = 80
B = 128
EW = CH * B
EPAD = NW * EW
RPT = MP // 16
_U = 8


@functools.cache
def _make_deg():
    mesh = plsc.VectorSubcoreMesh(core_axis_name="c", subcore_axis_name="s")

    @functools.partial(
        pl.kernel,
        out_type=jax.ShapeDtypeStruct((2, MP), jnp.float32),
        mesh=mesh,
        scratch_types=[
            pltpu.VMEM_SHARED((MP,), jnp.float32),
            pltpu.VMEM((CH, B), jnp.int32),
            pltpu.VMEM((B,), jnp.float32),
            pltpu.SemaphoreType.DMA,
        ],
    )
    def _deg(dst_hbm, zero_hbm, ones_hbm, out_hbm, acc_sh, idx_v, ones_v, sem):
        c = lax.axis_index("c")
        s = lax.axis_index("s")
        wid = s * 2 + c
        sl = pl.ds(s * RPT, RPT)
        pltpu.sync_copy(zero_hbm.at[sl], acc_sh.at[sl])
        pltpu.sync_copy(ones_hbm, ones_v)
        pltpu.sync_copy(dst_hbm.at[wid], idx_v)
        plsc.subcore_barrier()

        def group(jj):
            base = jj * _U
            ds = [pltpu.async_copy(ones_v, acc_sh.at[idx_v.at[base + b]],
                                   sem, add=True)
                  for b in range(_U)]
            for d in ds:
                d.wait()

        pl.loop(0, CH // _U)(group)
        plsc.subcore_barrier()
        pltpu.sync_copy(acc_sh.at[sl], out_hbm.at[c, sl])

    return _deg


@functools.cache
def _make_agg(w):
    mesh = plsc.VectorSubcoreMesh(core_axis_name="c", subcore_axis_name="s")

    @functools.partial(
        pl.kernel,
        out_type=jax.ShapeDtypeStruct((2, MP, w), jnp.float32),
        mesh=mesh,
        compiler_params=pltpu.CompilerParams(use_tc_tiling_on_sc=False),
        scratch_types=[
            pltpu.VMEM_SHARED((MP, w), jnp.float32),
            pltpu.VMEM_SHARED((MP, w), jnp.float32),
            pltpu.VMEM((CH, B), jnp.int32),
            pltpu.VMEM((CH, B), jnp.int32),
            pltpu.VMEM((_U, B, w), jnp.float32),
            pltpu.SemaphoreType.DMA((_U,)),
            pltpu.SemaphoreType.DMA((_U,)),
        ],
    )
    def _agg(g_hbm, src_hbm, dst_hbm, zero_hbm, out_hbm,
             g_sh, acc_sh, src_v, dst_v, rows_v, gsems, ssems):
        c = lax.axis_index("c")
        s = lax.axis_index("s")
        wid = s * 2 + c
        sl = pl.ds(s * RPT, RPT)
        pltpu.sync_copy(g_hbm.at[sl], g_sh.at[sl])
        pltpu.sync_copy(zero_hbm.at[sl], acc_sh.at[sl])
        pltpu.sync_copy(src_hbm.at[wid], src_v)
        pltpu.sync_copy(dst_hbm.at[wid], dst_v)
        plsc.subcore_barrier()

        ngrp = CH // _U

        def group(jj):
            base = jj * _U
            gds = []
            for b in range(_U):
                @pl.when(jj > 0)
                def _(b=b):
                    pltpu.make_async_copy(
                        rows_v.at[b],
                        acc_sh.at[dst_v.at[base - _U + b]],
                        ssems.at[b]).wait()
                gds.append(pltpu.async_copy(g_sh.at[src_v.at[base + b]],
                                            rows_v.at[b], gsems.at[b]))
            for b in range(_U):
                gds[b].wait()
                pltpu.async_copy(rows_v.at[b],
                                 acc_sh.at[dst_v.at[base + b]],
                                 ssems.at[b], add=True)

        pl.loop(0, ngrp)(group)
        for b in range(_U):
            pltpu.make_async_copy(
                rows_v.at[b],
                acc_sh.at[dst_v.at[(ngrp - 1) * _U + b]],
                ssems.at[b]).wait()
        plsc.subcore_barrier()
        pltpu.sync_copy(acc_sh.at[sl], out_hbm.at[c, sl])

    return _agg


def _relu6(v):
    return jnp.clip(v, 0.0, 6.0)


def _softmax(v):
    m = jnp.max(v, axis=-1, keepdims=True)
    e = jnp.exp(v - m)
    return e / jnp.sum(e, axis=-1, keepdims=True)


def _g1_body(nf_ref, w1_ref, degp_ref, g1_ref, dinv_ref):
    deg = degp_ref[0, :, :] + degp_ref[1, :, :] + 1.0
    r = lax.broadcasted_iota(jnp.int32, (MP, 1), 0)
    mask = (r < M).astype(jnp.float32)
    dinv = mask / jnp.sqrt(jnp.maximum(deg, 1.0))
    dinv_ref[...] = dinv
    h = jnp.dot(nf_ref[...], w1_ref[...], preferred_element_type=jnp.float32)
    g1_ref[...] = h * dinv


_g1_call = pl.pallas_call(
    _g1_body,
    out_shape=(jax.ShapeDtypeStruct((MP, 16), jnp.float32),
               jax.ShapeDtypeStruct((MP, 1), jnp.float32)))


def _make_mid(w_in, w_out):
    def _body(accp_ref, g_ref, dinv_ref, b_ref, w_ref, out_ref):
        acc = accp_ref[0, :, :] + accp_ref[1, :, :] + g_ref[...]
        dinv = dinv_ref[...]
        r = lax.broadcasted_iota(jnp.int32, (MP, 1), 0)
        mask = (r < M).astype(jnp.float32)
        nf = (acc * dinv + b_ref[...]) * mask
        h = jnp.dot(nf, w_ref[...], preferred_element_type=jnp.float32)
        out_ref[...] = h * dinv

    return pl.pallas_call(
        _body, out_shape=jax.ShapeDtypeStruct((MP, w_out), jnp.float32))


_mid1_call = _make_mid(16, 24)
_mid2_call = _make_mid(24, 32)


_RB = 1024
_NRB = MP // _RB


def _probs_body(accp_ref, g_ref, dinv_ref, b3_ref,
                ws1_ref, bs1_ref, ws2_ref, bs2_ref,
                we1_ref, be1_ref, we2_ref, be2_ref,
                nf3_ref, sprob_ref, eprob_ref, ps_ref, pe_ref):
    i = pl.program_id(0)
    acc = accp_ref[0, :, :] + accp_ref[1, :, :] + g_ref[...]
    dinv = dinv_ref[...]
    r = i * _RB + lax.broadcasted_iota(jnp.int32, (_RB, 1), 0)
    mask = (r < M).astype(jnp.float32)
    nf3 = (acc * dinv + b3_ref[...]) * mask
    nf3_ref[...] = nf3

    sh = _relu6(jnp.dot(nf3, ws1_ref[...], preferred_element_type=jnp.float32)
                + bs1_ref[...])
    slog = jnp.dot(sh, ws2_ref[...], preferred_element_type=jnp.float32) \
        + bs2_ref[...]
    sp = _softmax(slog) * (r < N_REAL).astype(jnp.float32)
    sprob_ref[...] = sp
    ps_ref[...] = jnp.sum(sp, axis=-1, keepdims=True)

    eh = _relu6(jnp.dot(nf3, we1_ref[...], preferred_element_type=jnp.float32)
                + be1_ref[...])
    elog = jnp.dot(eh, we2_ref[...], preferred_element_type=jnp.float32) \
        + be2_ref[...]
    ep = _softmax(elog)
    eprob_ref[...] = ep
    pe_ref[...] = jnp.sum(ep, axis=-1, keepdims=True)


def _full_spec(shape):
    return pl.BlockSpec(shape, lambda i: tuple(0 for _ in shape))


_probs_call = pl.pallas_call(
    _probs_body,
    grid=(_NRB,),
    in_specs=[
        pl.BlockSpec((2, _RB, 32), lambda i: (0, i, 0)),
        pl.BlockSpec((_RB, 32), lambda i: (i, 0)),
        pl.BlockSpec((_RB, 1), lambda i: (i, 0)),
        _full_spec((32,)),
        _full_spec((32, 16)), _full_spec((16,)),
        _full_spec((16, 10)), _full_spec((10,)),
        _full_spec((32, 24)), _full_spec((24,)),
        _full_spec((24, 10)), _full_spec((10,)),
    ],
    out_specs=(
        pl.BlockSpec((_RB, 32), lambda i: (i, 0)),
        pl.BlockSpec((_RB, 10), lambda i: (i, 0)),
        pl.BlockSpec((_RB, 10), lambda i: (i, 0)),
        pl.BlockSpec((_RB, 1), lambda i: (i, 0)),
        pl.BlockSpec((_RB, 1), lambda i: (i, 0)),
    ),
    out_shape=(
        jax.ShapeDtypeStruct((MP, 32), jnp.float32),
        jax.ShapeDtypeStruct((MP, 10), jnp.float32),
        jax.ShapeDtypeStruct((MP, 10), jnp.float32),
        jax.ShapeDtypeStruct((MP, 1), jnp.float32),
        jax.ShapeDtypeStruct((MP, 1), jnp.float32),
    ),
)


def _sample_body(nf3_ref, ep_ref, ps_ref, pe_ref,
                 we1_ref, be1_ref, we2_ref, be2_ref,
                 g42_ref, g43_ref,
                 eprob_ref, start_ref, end_ref):
    r = lax.broadcasted_iota(jnp.int32, (MP, 1), 0)
    score_s = jnp.log(ps_ref[...] + 1e-12) + g42_ref[...]
    best_s = jnp.max(score_s)
    sidx = jnp.min(jnp.where(score_s == best_s, r, MP))
    start_ref[...] = jnp.broadcast_to(sidx, (1, 1))

    nf3 = nf3_ref[...]
    onehot = (lax.broadcasted_iota(jnp.int32, (1, MP), 1) == sidx
              ).astype(jnp.float32)
    row = jnp.dot(onehot, nf3, preferred_element_type=jnp.float32)
    ehr = _relu6(jnp.dot(row, we1_ref[...], preferred_element_type=jnp.float32)
                 + be1_ref[...])
    elr = jnp.dot(ehr, we2_ref[...], preferred_element_type=jnp.float32) \
        + be2_ref[...]
    epr = _softmax(elr)

    is_start = (r == sidx)
    is_extra = (r == M)
    ep_out = jnp.where(is_start, 0.0, jnp.where(is_extra, epr, ep_ref[...]))
    eprob_ref[...] = ep_out
    p_end = jnp.where(is_extra, jnp.sum(epr), pe_ref[...])
    p_end = jnp.where(is_start | (r > M), 0.0, p_end)
    score_e = jnp.log(p_end + 1e-12) + g43_ref[...]
    best_e = jnp.max(score_e)
    end_ref[...] = jnp.broadcast_to(
        jnp.min(jnp.where(score_e == best_e, r, MP)), (1, 1))


_sample_call = pl.pallas_call(
    _sample_body,
    out_shape=(
        jax.ShapeDtypeStruct((MP, 10), jnp.float32),
        jax.ShapeDtypeStruct((1, 1), jnp.int32),
        jax.ShapeDtypeStruct((1, 1), jnp.int32),
    ),
)


def kernel(x, edge_index, candidate_set, W1, b1, W2, b2, W3, b3,
           Ws1, bs1, Ws2, bs2, We1, be1, We2, be2):
    f32 = jnp.float32
    nf = jnp.zeros((MP, 128), f32)
    nf = nf.at[:N_REAL].set(x).at[N_REAL:M].set(candidate_set)

    pad = EPAD - E
    pad_idx = M + (jnp.arange(pad, dtype=jnp.int32) % (MP - M))
    src_p = jnp.concatenate([edge_index[0], pad_idx]).reshape(NW, CH, B)
    dst_p = jnp.concatenate([edge_index[1], pad_idx]).reshape(NW, CH, B)

    zeros1 = jnp.zeros((MP,), f32)
    zeros16 = jnp.zeros((MP, 16), f32)
    zeros24 = jnp.zeros((MP, 24), f32)
    zeros32 = jnp.zeros((MP, 32), f32)
    ones128 = jnp.ones((B,), f32)

    gum_s = jax.random.gumbel(jax.random.key(42), (M,), f32)
    gum_e = jax.random.gumbel(jax.random.key(43), (M + 1,), f32)
    g42 = jnp.full((MP,), -1e30, f32).at[:M].set(gum_s).reshape(MP, 1)
    g43 = jnp.full((MP,), -1e30, f32).at[:M + 1].set(gum_e).reshape(MP, 1)

    degp = _make_deg()(dst_p, zeros1, ones128)
    g1, dinv = _g1_call(nf, W1, degp.reshape(2, MP, 1))
    accp1 = _make_agg(16)(g1, src_p, dst_p, zeros16)
    g2 = _mid1_call(accp1, g1, dinv, b1, W2)
    accp2 = _make_agg(24)(g2, src_p, dst_p, zeros24)
    g3 = _mid2_call(accp2, g2, dinv, b2, W3)
    accp3 = _make_agg(32)(g3, src_p, dst_p, zeros32)

    nf3, sprob, ep_raw, p_start, p_end = _probs_call(
        accp3, g3, dinv, b3, Ws1, bs1, Ws2, bs2, We1, be1, We2, be2)
    eprob, start2, end2 = _sample_call(
        nf3, ep_raw, p_start, p_end, We1, be1, We2, be2, g42, g43)

    start_node = start2.reshape(())
    end_node = end2.reshape(())
    return (start_node, end_node, sprob[:M], eprob[:M + 1])


# speedup vs baseline: 1.0385x; 1.0385x over previous
import functools

import jax
import jax.numpy as jnp
from jax import lax
from jax.experimental import pallas as pl
from jax.experimental.pallas import tpu as pltpu
from jax.experimental.pallas import tpu_sc as plsc

N_REAL = 10000
N_CANDS = 100
M = N_REAL + N_CANDS
MP = 10240
E = 320000
NW = 32
CH = 80
B = 128
EW = CH * B
EPAD = NW * EW
RPT = MP // 16
_U = 8


@functools.cache
def _make_deg():
    mesh = plsc.VectorSubcoreMesh(core_axis_name="c", subcore_axis_name="s")

    @functools.partial(
        pl.kernel,
        out_type=jax.ShapeDtypeStruct((2, MP), jnp.float32),
        mesh=mesh,
        scratch_types=[
            pltpu.VMEM_SHARED((MP,), jnp.float32),
            pltpu.VMEM((CH, B), jnp.int32),
            pltpu.VMEM((B,), jnp.float32),
            pltpu.SemaphoreType.DMA,
        ],
    )
    def _deg(dst_hbm, zero_hbm, ones_hbm, out_hbm, acc_sh, idx_v, ones_v, sem):
        c = lax.axis_index("c")
        s = lax.axis_index("s")
        wid = s * 2 + c
        sl = pl.ds(s * RPT, RPT)
        pltpu.sync_copy(zero_hbm.at[sl], acc_sh.at[sl])
        pltpu.sync_copy(ones_hbm, ones_v)
        pltpu.sync_copy(dst_hbm.at[wid], idx_v)
        plsc.subcore_barrier()

        def group(jj):
            base = jj * _U
            ds = [pltpu.async_copy(ones_v, acc_sh.at[idx_v.at[base + b]],
                                   sem, add=True)
                  for b in range(_U)]
            for d in ds:
                d.wait()

        pl.loop(0, CH // _U)(group)
        plsc.subcore_barrier()
        pltpu.sync_copy(acc_sh.at[sl], out_hbm.at[c, sl])

    return _deg


@functools.cache
def _make_agg(w):
    mesh = plsc.VectorSubcoreMesh(core_axis_name="c", subcore_axis_name="s")

    @functools.partial(
        pl.kernel,
        out_type=jax.ShapeDtypeStruct((2, MP, w), jnp.float32),
        mesh=mesh,
        compiler_params=pltpu.CompilerParams(use_tc_tiling_on_sc=False),
        scratch_types=[
            pltpu.VMEM_SHARED((MP, w), jnp.float32),
            pltpu.VMEM((CH, B), jnp.int32),
            pltpu.VMEM((CH, B), jnp.int32),
            pltpu.VMEM((_U, B, w), jnp.float32),
            pltpu.SemaphoreType.DMA((_U,)),
            pltpu.SemaphoreType.DMA((_U,)),
        ],
    )
    def _agg(g_hbm, src_hbm, dst_hbm, zero_hbm, out_hbm,
             acc_sh, src_v, dst_v, rows_v, gsems, ssems):
        c = lax.axis_index("c")
        s = lax.axis_index("s")
        wid = s * 2 + c
        sl = pl.ds(s * RPT, RPT)
        pltpu.sync_copy(zero_hbm.at[sl], acc_sh.at[sl])
        pltpu.sync_copy(src_hbm.at[wid], src_v)
        pltpu.sync_copy(dst_hbm.at[wid], dst_v)
        plsc.subcore_barrier()

        ngrp = CH // _U

        def group(jj):
            base = jj * _U
            gds = []
            for b in range(_U):
                @pl.when(jj > 0)
                def _(b=b):
                    pltpu.make_async_copy(
                        rows_v.at[b],
                        acc_sh.at[dst_v.at[base - _U + b]],
                        ssems.at[b]).wait()
                gds.append(pltpu.async_copy(g_hbm.at[src_v.at[base + b]],
                                            rows_v.at[b], gsems.at[b]))
            for b in range(_U):
                gds[b].wait()
                pltpu.async_copy(rows_v.at[b],
                                 acc_sh.at[dst_v.at[base + b]],
                                 ssems.at[b], add=True)

        pl.loop(0, ngrp)(group)
        for b in range(_U):
            pltpu.make_async_copy(
                rows_v.at[b],
                acc_sh.at[dst_v.at[(ngrp - 1) * _U + b]],
                ssems.at[b]).wait()
        plsc.subcore_barrier()
        pltpu.sync_copy(acc_sh.at[sl], out_hbm.at[c, sl])

    return _agg


def _relu6(v):
    return jnp.clip(v, 0.0, 6.0)


def _softmax(v):
    m = jnp.max(v, axis=-1, keepdims=True)
    e = jnp.exp(v - m)
    return e / jnp.sum(e, axis=-1, keepdims=True)


def _g1_body(nf_ref, w1_ref, degp_ref, g1_ref, dinv_ref):
    deg = degp_ref[0, :, :] + degp_ref[1, :, :] + 1.0
    r = lax.broadcasted_iota(jnp.int32, (MP, 1), 0)
    mask = (r < M).astype(jnp.float32)
    dinv = mask / jnp.sqrt(jnp.maximum(deg, 1.0))
    dinv_ref[...] = dinv
    h = jnp.dot(nf_ref[...], w1_ref[...], preferred_element_type=jnp.float32)
    g1_ref[...] = h * dinv


_g1_call = pl.pallas_call(
    _g1_body,
    out_shape=(jax.ShapeDtypeStruct((MP, 16), jnp.float32),
               jax.ShapeDtypeStruct((MP, 1), jnp.float32)))


def _make_mid(w_in, w_out):
    def _body(accp_ref, g_ref, dinv_ref, b_ref, w_ref, out_ref):
        acc = accp_ref[0, :, :] + accp_ref[1, :, :] + g_ref[...]
        dinv = dinv_ref[...]
        r = lax.broadcasted_iota(jnp.int32, (MP, 1), 0)
        mask = (r < M).astype(jnp.float32)
        nf = (acc * dinv + b_ref[...]) * mask
        h = jnp.dot(nf, w_ref[...], preferred_element_type=jnp.float32)
        out_ref[...] = h * dinv

    return pl.pallas_call(
        _body, out_shape=jax.ShapeDtypeStruct((MP, w_out), jnp.float32))


_mid1_call = _make_mid(16, 24)
_mid2_call = _make_mid(24, 32)


_RB = 1024
_NRB = MP // _RB


def _probs_body(accp_ref, g_ref, dinv_ref, b3_ref,
                ws1_ref, bs1_ref, ws2_ref, bs2_ref,
                we1_ref, be1_ref, we2_ref, be2_ref,
                nf3_ref, sprob_ref, eprob_ref, ps_ref, pe_ref):
    i = pl.program_id(0)
    acc = accp_ref[0, :, :] + accp_ref[1, :, :] + g_ref[...]
    dinv = dinv_ref[...]
    r = i * _RB + lax.broadcasted_iota(jnp.int32, (_RB, 1), 0)
    mask = (r < M).astype(jnp.float32)
    nf3 = (acc * dinv + b3_ref[...]) * mask
    nf3_ref[...] = nf3

    sh = _relu6(jnp.dot(nf3, ws1_ref[...], preferred_element_type=jnp.float32)
                + bs1_ref[...])
    slog = jnp.dot(sh, ws2_ref[...], preferred_element_type=jnp.float32) \
        + bs2_ref[...]
    sp = _softmax(slog) * (r < N_REAL).astype(jnp.float32)
    sprob_ref[...] = sp
    ps_ref[...] = jnp.sum(sp, axis=-1, keepdims=True)

    eh = _relu6(jnp.dot(nf3, we1_ref[...], preferred_element_type=jnp.float32)
                + be1_ref[...])
    elog = jnp.dot(eh, we2_ref[...], preferred_element_type=jnp.float32) \
        + be2_ref[...]
    ep = _softmax(elog)
    eprob_ref[...] = ep
    pe_ref[...] = jnp.sum(ep, axis=-1, keepdims=True)


def _full_spec(shape):
    return pl.BlockSpec(shape, lambda i: tuple(0 for _ in shape))


_probs_call = pl.pallas_call(
    _probs_body,
    grid=(_NRB,),
    in_specs=[
        pl.BlockSpec((2, _RB, 32), lambda i: (0, i, 0)),
        pl.BlockSpec((_RB, 32), lambda i: (i, 0)),
        pl.BlockSpec((_RB, 1), lambda i: (i, 0)),
        _full_spec((32,)),
        _full_spec((32, 16)), _full_spec((16,)),
        _full_spec((16, 10)), _full_spec((10,)),
        _full_spec((32, 24)), _full_spec((24,)),
        _full_spec((24, 10)), _full_spec((10,)),
    ],
    out_specs=(
        pl.BlockSpec((_RB, 32), lambda i: (i, 0)),
        pl.BlockSpec((_RB, 10), lambda i: (i, 0)),
        pl.BlockSpec((_RB, 10), lambda i: (i, 0)),
        pl.BlockSpec((_RB, 1), lambda i: (i, 0)),
        pl.BlockSpec((_RB, 1), lambda i: (i, 0)),
    ),
    out_shape=(
        jax.ShapeDtypeStruct((MP, 32), jnp.float32),
        jax.ShapeDtypeStruct((MP, 10), jnp.float32),
        jax.ShapeDtypeStruct((MP, 10), jnp.float32),
        jax.ShapeDtypeStruct((MP, 1), jnp.float32),
        jax.ShapeDtypeStruct((MP, 1), jnp.float32),
    ),
)


def _sample_body(nf3_ref, ep_ref, ps_ref, pe_ref,
                 we1_ref, be1_ref, we2_ref, be2_ref,
                 g42_ref, g43_ref,
                 eprob_ref, start_ref, end_ref):
    r = lax.broadcasted_iota(jnp.int32, (MP, 1), 0)
    score_s = jnp.log(ps_ref[...] + 1e-12) + g42_ref[...]
    best_s = jnp.max(score_s)
    sidx = jnp.min(jnp.where(score_s == best_s, r, MP))
    start_ref[...] = jnp.broadcast_to(sidx, (1, 1))

    nf3 = nf3_ref[...]
    onehot = (lax.broadcasted_iota(jnp.int32, (1, MP), 1) == sidx
              ).astype(jnp.float32)
    row = jnp.dot(onehot, nf3, preferred_element_type=jnp.float32)
    ehr = _relu6(jnp.dot(row, we1_ref[...], preferred_element_type=jnp.float32)
                 + be1_ref[...])
    elr = jnp.dot(ehr, we2_ref[...], preferred_element_type=jnp.float32) \
        + be2_ref[...]
    epr = _softmax(elr)

    is_start = (r == sidx)
    is_extra = (r == M)
    ep_out = jnp.where(is_start, 0.0, jnp.where(is_extra, epr, ep_ref[...]))
    eprob_ref[...] = ep_out
    p_end = jnp.where(is_extra, jnp.sum(epr), pe_ref[...])
    p_end = jnp.where(is_start | (r > M), 0.0, p_end)
    score_e = jnp.log(p_end + 1e-12) + g43_ref[...]
    best_e = jnp.max(score_e)
    end_ref[...] = jnp.broadcast_to(
        jnp.min(jnp.where(score_e == best_e, r, MP)), (1, 1))


_sample_call = pl.pallas_call(
    _sample_body,
    out_shape=(
        jax.ShapeDtypeStruct((MP, 10), jnp.float32),
        jax.ShapeDtypeStruct((1, 1), jnp.int32),
        jax.ShapeDtypeStruct((1, 1), jnp.int32),
    ),
)


def kernel(x, edge_index, candidate_set, W1, b1, W2, b2, W3, b3,
           Ws1, bs1, Ws2, bs2, We1, be1, We2, be2):
    f32 = jnp.float32
    nf = jnp.zeros((MP, 128), f32)
    nf = nf.at[:N_REAL].set(x).at[N_REAL:M].set(candidate_set)

    pad = EPAD - E
    pad_idx = M + (jnp.arange(pad, dtype=jnp.int32) % (MP - M))
    src_p = jnp.concatenate([edge_index[0], pad_idx]).reshape(NW, CH, B)
    dst_p = jnp.concatenate([edge_index[1], pad_idx]).reshape(NW, CH, B)

    zeros1 = jnp.zeros((MP,), f32)
    zeros16 = jnp.zeros((MP, 16), f32)
    zeros24 = jnp.zeros((MP, 24), f32)
    zeros32 = jnp.zeros((MP, 32), f32)
    ones128 = jnp.ones((B,), f32)

    gum_s = jax.random.gumbel(jax.random.key(42), (M,), f32)
    gum_e = jax.random.gumbel(jax.random.key(43), (M + 1,), f32)
    g42 = jnp.full((MP,), -1e30, f32).at[:M].set(gum_s).reshape(MP, 1)
    g43 = jnp.full((MP,), -1e30, f32).at[:M + 1].set(gum_e).reshape(MP, 1)

    degp = _make_deg()(dst_p, zeros1, ones128)
    g1, dinv = _g1_call(nf, W1, degp.reshape(2, MP, 1))
    accp1 = _make_agg(16)(g1, src_p, dst_p, zeros16)
    g2 = _mid1_call(accp1, g1, dinv, b1, W2)
    accp2 = _make_agg(24)(g2, src_p, dst_p, zeros24)
    g3 = _mid2_call(accp2, g2, dinv, b2, W3)
    accp3 = _make_agg(32)(g3, src_p, dst_p, zeros32)

    nf3, sprob, ep_raw, p_start, p_end = _probs_call(
        accp3, g3, dinv, b3, Ws1, bs1, Ws2, bs2, We1, be1, We2, be2)
    eprob, start2, end2 = _sample_call(
        nf3, ep_raw, p_start, p_end, We1, be1, We2, be2, g42, g43)

    start_node = start2.reshape(())
    end_node = end2.reshape(())
    return (start_node, end_node, sprob[:M], eprob[:M + 1])


# speedup vs baseline: 1.1417x; 1.0994x over previous
import functools

import jax
import jax.numpy as jnp
from jax import lax
from jax.experimental import pallas as pl
from jax.experimental.pallas import tpu as pltpu
from jax.experimental.pallas import tpu_sc as plsc

N_REAL = 10000
N_CANDS = 100
M = N_REAL + N_CANDS
MP = 10240
E = 320000
NW = 32
CH = 80
B = 128
EW = CH * B
EPAD = NW * EW
RPT = MP // 16
_U = 8


@functools.cache
def _make_deg():
    mesh = plsc.VectorSubcoreMesh(core_axis_name="c", subcore_axis_name="s")

    @functools.partial(
        pl.kernel,
        out_type=jax.ShapeDtypeStruct((2, MP), jnp.float32),
        mesh=mesh,
        scratch_types=[
            pltpu.VMEM_SHARED((MP,), jnp.float32),
            pltpu.VMEM((CH, B), jnp.int32),
            pltpu.VMEM((B,), jnp.float32),
            pltpu.SemaphoreType.DMA,
        ],
    )
    def _deg(dst_hbm, zero_hbm, ones_hbm, out_hbm, acc_sh, idx_v, ones_v, sem):
        c = lax.axis_index("c")
        s = lax.axis_index("s")
        wid = s * 2 + c
        sl = pl.ds(s * RPT, RPT)
        pltpu.sync_copy(zero_hbm.at[sl], acc_sh.at[sl])
        pltpu.sync_copy(ones_hbm, ones_v)
        pltpu.sync_copy(dst_hbm.at[wid], idx_v)
        plsc.subcore_barrier()

        def group(jj):
            base = jj * _U
            ds = [pltpu.async_copy(ones_v, acc_sh.at[idx_v.at[base + b]],
                                   sem, add=True)
                  for b in range(_U)]
            for d in ds:
                d.wait()

        pl.loop(0, CH // _U)(group)
        plsc.subcore_barrier()
        pltpu.sync_copy(acc_sh.at[sl], out_hbm.at[c, sl])

    return _deg


@functools.cache
def _make_agg(w):
    mesh = plsc.VectorSubcoreMesh(core_axis_name="c", subcore_axis_name="s")

    @functools.partial(
        pl.kernel,
        out_type=jax.ShapeDtypeStruct((2, MP, w), jnp.float32),
        mesh=mesh,
        compiler_params=pltpu.CompilerParams(use_tc_tiling_on_sc=False),
        scratch_types=[
            pltpu.VMEM_SHARED((MP, w), jnp.float32),
            pltpu.VMEM((CH, B), jnp.int32),
            pltpu.VMEM((CH, B), jnp.int32),
            pltpu.VMEM((_U, B, w), jnp.float32),
            pltpu.SemaphoreType.DMA((_U,)),
            pltpu.SemaphoreType.DMA((_U,)),
        ],
    )
    def _agg(g_hbm, src_hbm, dst_hbm, zero_hbm, out_hbm,
             acc_sh, src_v, dst_v, rows_v, gsems, ssems):
        c = lax.axis_index("c")
        s = lax.axis_index("s")
        wid = s * 2 + c
        sl = pl.ds(s * RPT, RPT)
        pltpu.sync_copy(zero_hbm.at[sl], acc_sh.at[sl])
        pltpu.sync_copy(src_hbm.at[wid], src_v)
        pltpu.sync_copy(dst_hbm.at[wid], dst_v)
        plsc.subcore_barrier()

        ngrp = CH // _U

        def group(jj):
            base = jj * _U
            gds = []
            for b in range(_U):
                @pl.when(jj > 0)
                def _(b=b):
                    pltpu.make_async_copy(
                        rows_v.at[b],
                        acc_sh.at[dst_v.at[base - _U + b]],
                        ssems.at[b]).wait()
                gds.append(pltpu.async_copy(g_hbm.at[src_v.at[base + b]],
                                            rows_v.at[b], gsems.at[b]))
            for b in range(_U):
                gds[b].wait()
                pltpu.async_copy(rows_v.at[b],
                                 acc_sh.at[dst_v.at[base + b]],
                                 ssems.at[b], add=True)

        pl.loop(0, ngrp)(group)
        for b in range(_U):
            pltpu.make_async_copy(
                rows_v.at[b],
                acc_sh.at[dst_v.at[(ngrp - 1) * _U + b]],
                ssems.at[b]).wait()
        plsc.subcore_barrier()
        pltpu.sync_copy(acc_sh.at[sl], out_hbm.at[c, sl])

    return _agg


def _relu6(v):
    return jnp.clip(v, 0.0, 6.0)


def _softmax(v):
    m = jnp.max(v, axis=-1, keepdims=True)
    e = jnp.exp(v - m)
    return e / jnp.sum(e, axis=-1, keepdims=True)


def _g1_body(x_ref, tail_ref, w1_ref, degp_ref, g1_ref, dinv_ref):
    deg = degp_ref[0, :, :] + degp_ref[1, :, :] + 1.0
    r = lax.broadcasted_iota(jnp.int32, (MP, 1), 0)
    mask = (r < M).astype(jnp.float32)
    dinv = mask / jnp.sqrt(jnp.maximum(deg, 1.0))
    dinv_ref[...] = dinv
    w1 = w1_ref[...]
    h_main = jnp.dot(x_ref[...], w1, preferred_element_type=jnp.float32)
    h_tail = jnp.dot(tail_ref[...], w1, preferred_element_type=jnp.float32)
    g1_ref[:N_REAL, :] = h_main * dinv[:N_REAL, :]
    g1_ref[N_REAL:, :] = h_tail * dinv[N_REAL:, :]


_g1_call = pl.pallas_call(
    _g1_body,
    out_shape=(jax.ShapeDtypeStruct((MP, 16), jnp.float32),
               jax.ShapeDtypeStruct((MP, 1), jnp.float32)))


def _make_mid(w_in, w_out):
    def _body(accp_ref, g_ref, dinv_ref, b_ref, w_ref, out_ref):
        acc = accp_ref[0, :, :] + accp_ref[1, :, :] + g_ref[...]
        dinv = dinv_ref[...]
        r = lax.broadcasted_iota(jnp.int32, (MP, 1), 0)
        mask = (r < M).astype(jnp.float32)
        nf = (acc * dinv + b_ref[...]) * mask
        h = jnp.dot(nf, w_ref[...], preferred_element_type=jnp.float32)
        out_ref[...] = h * dinv

    return pl.pallas_call(
        _body, out_shape=jax.ShapeDtypeStruct((MP, w_out), jnp.float32))


_mid1_call = _make_mid(16, 24)
_mid2_call = _make_mid(24, 32)


_RB = 2048
_NRB = MP // _RB


def _probs_body(accp_ref, g_ref, dinv_ref, b3_ref,
                ws1_ref, bs1_ref, ws2_ref, bs2_ref,
                we1_ref, be1_ref, we2_ref, be2_ref,
                nf3_ref, sprob_ref, eprob_ref, ps_ref, pe_ref):
    i = pl.program_id(0)
    acc = accp_ref[0, :, :] + accp_ref[1, :, :] + g_ref[...]
    dinv = dinv_ref[...]
    r = i * _RB + lax.broadcasted_iota(jnp.int32, (_RB, 1), 0)
    mask = (r < M).astype(jnp.float32)
    nf3 = (acc * dinv + b3_ref[...]) * mask
    nf3_ref[...] = nf3

    sh = _relu6(jnp.dot(nf3, ws1_ref[...], preferred_element_type=jnp.float32)
                + bs1_ref[...])
    slog = jnp.dot(sh, ws2_ref[...], preferred_element_type=jnp.float32) \
        + bs2_ref[...]
    sp = _softmax(slog) * (r < N_REAL).astype(jnp.float32)
    sprob_ref[...] = sp
    ps_ref[...] = jnp.sum(sp, axis=-1, keepdims=True)

    eh = _relu6(jnp.dot(nf3, we1_ref[...], preferred_element_type=jnp.float32)
                + be1_ref[...])
    elog = jnp.dot(eh, we2_ref[...], preferred_element_type=jnp.float32) \
        + be2_ref[...]
    ep = _softmax(elog)
    eprob_ref[...] = ep
    pe_ref[...] = jnp.sum(ep, axis=-1, keepdims=True)


def _full_spec(shape):
    return pl.BlockSpec(shape, lambda i: tuple(0 for _ in shape))


_probs_call = pl.pallas_call(
    _probs_body,
    grid=(_NRB,),
    in_specs=[
        pl.BlockSpec((2, _RB, 32), lambda i: (0, i, 0)),
        pl.BlockSpec((_RB, 32), lambda i: (i, 0)),
        pl.BlockSpec((_RB, 1), lambda i: (i, 0)),
        _full_spec((32,)),
        _full_spec((32, 16)), _full_spec((16,)),
        _full_spec((16, 10)), _full_spec((10,)),
        _full_spec((32, 24)), _full_spec((24,)),
        _full_spec((24, 10)), _full_spec((10,)),
    ],
    out_specs=(
        pl.BlockSpec((_RB, 32), lambda i: (i, 0)),
        pl.BlockSpec((_RB, 10), lambda i: (i, 0)),
        pl.BlockSpec((_RB, 10), lambda i: (i, 0)),
        pl.BlockSpec((_RB, 1), lambda i: (i, 0)),
        pl.BlockSpec((_RB, 1), lambda i: (i, 0)),
    ),
    out_shape=(
        jax.ShapeDtypeStruct((MP, 32), jnp.float32),
        jax.ShapeDtypeStruct((MP, 10), jnp.float32),
        jax.ShapeDtypeStruct((MP, 10), jnp.float32),
        jax.ShapeDtypeStruct((MP, 1), jnp.float32),
        jax.ShapeDtypeStruct((MP, 1), jnp.float32),
    ),
)


_RD = MP // 128


def _sample_body(nf3_ref, ep_ref, ps_ref, pe_ref,
                 we1_ref, be1_ref, we2_ref, be2_ref,
                 g42_ref, g43_ref,
                 eprob_ref, start_ref, end_ref):
    flat = (lax.broadcasted_iota(jnp.int32, (_RD, 128), 0) * 128
            + lax.broadcasted_iota(jnp.int32, (_RD, 128), 1))
    score_s = jnp.log(ps_ref[...] + 1e-12) + g42_ref[...]
    best_s = jnp.max(score_s)
    sidx = jnp.min(jnp.where(score_s == best_s, flat, MP))
    start_ref[...] = jnp.broadcast_to(sidx, (1, 1))

    row = nf3_ref[pl.ds(sidx, 1), :]
    ehr = _relu6(jnp.dot(row, we1_ref[...], preferred_element_type=jnp.float32)
                 + be1_ref[...])
    elr = jnp.dot(ehr, we2_ref[...], preferred_element_type=jnp.float32) \
        + be2_ref[...]
    epr = _softmax(elr)

    r = lax.broadcasted_iota(jnp.int32, (MP, 1), 0)
    is_start = (r == sidx)
    is_extra = (r == M)
    ep_out = jnp.where(is_start, 0.0, jnp.where(is_extra, epr, ep_ref[...]))
    eprob_ref[...] = ep_out
    p_end = jnp.where(flat == M, jnp.sum(epr), pe_ref[...])
    p_end = jnp.where((flat == sidx) | (flat > M), 0.0, p_end)
    score_e = jnp.log(p_end + 1e-12) + g43_ref[...]
    best_e = jnp.max(score_e)
    end_ref[...] = jnp.broadcast_to(
        jnp.min(jnp.where(score_e == best_e, flat, MP)), (1, 1))


_sample_call = pl.pallas_call(
    _sample_body,
    out_shape=(
        jax.ShapeDtypeStruct((MP, 10), jnp.float32),
        jax.ShapeDtypeStruct((1, 1), jnp.int32),
        jax.ShapeDtypeStruct((1, 1), jnp.int32),
    ),
)


@functools.cache
def _gumbel_consts():
    f32 = jnp.float32
    gum_s = jax.random.gumbel(jax.random.key(42), (M,), f32)
    gum_e = jax.random.gumbel(jax.random.key(43), (M + 1,), f32)
    g42 = jnp.full((MP,), -1e30, f32).at[:M].set(gum_s).reshape(_RD, 128)
    g43 = jnp.full((MP,), -1e30, f32).at[:M + 1].set(gum_e).reshape(_RD, 128)
    return jax.block_until_ready(g42), jax.block_until_ready(g43)


def kernel(x, edge_index, candidate_set, W1, b1, W2, b2, W3, b3,
           Ws1, bs1, Ws2, bs2, We1, be1, We2, be2):
    f32 = jnp.float32
    tail = jnp.zeros((MP - N_REAL, 128), f32).at[:N_CANDS].set(candidate_set)

    pad = EPAD - E
    pad_idx = M + (jnp.arange(pad, dtype=jnp.int32) % (MP - M))
    src_p = jnp.concatenate([edge_index[0], pad_idx]).reshape(NW, CH, B)
    dst_p = jnp.concatenate([edge_index[1], pad_idx]).reshape(NW, CH, B)

    zeros1 = jnp.zeros((MP,), f32)
    zeros16 = jnp.zeros((MP, 16), f32)
    zeros24 = jnp.zeros((MP, 24), f32)
    zeros32 = jnp.zeros((MP, 32), f32)
    ones128 = jnp.ones((B,), f32)

    g42, g43 = _gumbel_consts()

    degp = _make_deg()(dst_p, zeros1, ones128)
    g1, dinv = _g1_call(x, tail, W1, degp.reshape(2, MP, 1))
    accp1 = _make_agg(16)(g1, src_p, dst_p, zeros16)
    g2 = _mid1_call(accp1, g1, dinv, b1, W2)
    accp2 = _make_agg(24)(g2, src_p, dst_p, zeros24)
    g3 = _mid2_call(accp2, g2, dinv, b2, W3)
    accp3 = _make_agg(32)(g3, src_p, dst_p, zeros32)

    nf3, sprob, ep_raw, p_start, p_end = _probs_call(
        accp3, g3, dinv, b3, Ws1, bs1, Ws2, bs2, We1, be1, We2, be2)
    eprob, start2, end2 = _sample_call(
        nf3, ep_raw, p_start.reshape(_RD, 128), p_end.reshape(_RD, 128),
        We1, be1, We2, be2, g42, g43)

    start_node = start2.reshape(())
    end_node = end2.reshape(())
    return (start_node, end_node, sprob[:M], eprob[:M + 1])
